# baseline (speedup 1.0000x reference)
import functools

import jax
import jax.numpy as jnp
from jax import lax
from jax.experimental import pallas as pl
from jax.experimental.pallas import tpu as pltpu
from jax.experimental.pallas import tpu_sc as plsc

B = 4096
V = 100000
K = 5
C = 160
NCHUNK = V // C
FLAT = B * NCHUNK

_AM = 4096
_BM_SEL = 512
_BM_LOSS = 512
NEG = float("-inf")


def _chunk_max(flat):
    def body(x_ref, o_ref):
        o_ref[...] = jnp.max(x_ref[...], axis=1, keepdims=True)

    return pl.pallas_call(
        body,
        grid=(FLAT // _AM,),
        in_specs=[pl.BlockSpec((_AM, C), lambda i: (i, 0))],
        out_specs=pl.BlockSpec((_AM, 1), lambda i: (i, 0)),
        out_shape=jax.ShapeDtypeStruct((FLAT, 1), jnp.float32),
    )(flat)


def _select_chunks(m, labels2):
    def body(m_ref, lab_ref, gidx_ref, lrow_ref, llane_ref):
        rb = pl.program_id(0)
        m = m_ref[...]
        colio = lax.broadcasted_iota(jnp.int32, (_BM_SEL, NCHUNK), 1)
        idxs = []
        for _ in range(K):
            mx = jnp.max(m, axis=1, keepdims=True)
            idx = jnp.min(jnp.where(m == mx, colio, NCHUNK), axis=1,
                          keepdims=True)
            idxs.append(idx)
            m = jnp.where(colio == idx, NEG, m)
        rowio = rb * _BM_SEL + lax.broadcasted_iota(
            jnp.int32, (_BM_SEL, 1), 0)
        base = rowio * NCHUNK
        gidx_ref[...] = jnp.concatenate([base + i for i in idxs], axis=1)
        lab = lab_ref[...]
        lrow_ref[...] = base + lab // C
        llane_ref[...] = lab % C

    return pl.pallas_call(
        body,
        grid=(B // _BM_SEL,),
        in_specs=[pl.BlockSpec((_BM_SEL, NCHUNK), lambda i: (i, 0)),
                  pl.BlockSpec((_BM_SEL, 1), lambda i: (i, 0))],
        out_specs=[pl.BlockSpec((_BM_SEL, K), lambda i: (i, 0)),
                   pl.BlockSpec((_BM_SEL, 1), lambda i: (i, 0)),
                   pl.BlockSpec((_BM_SEL, 1), lambda i: (i, 0))],
        out_shape=[jax.ShapeDtypeStruct((B, K), jnp.int32),
                   jax.ShapeDtypeStruct((B, 1), jnp.int32),
                   jax.ShapeDtypeStruct((B, 1), jnp.int32)],
    )(m, labels2)


def _sc_gather(table, gidx3, lrow2, nc, ns):
    nw = nc * ns
    rpt = B // nw

    mesh = plsc.VectorSubcoreMesh(core_axis_name="c", subcore_axis_name="s")

    @functools.partial(
        pl.kernel,
        mesh=mesh,
        compiler_params=pltpu.CompilerParams(use_tc_tiling_on_sc=False),
        out_type=[jax.ShapeDtypeStruct((B * K, C), jnp.float32),
                  jax.ShapeDtypeStruct((B, C), jnp.float32)],
        scratch_types=[
            pltpu.VMEM((K, rpt), jnp.int32),
            pltpu.VMEM((rpt,), jnp.int32),
            pltpu.VMEM((rpt, C), jnp.float32),
            pltpu.VMEM((rpt, C), jnp.float32),
            pltpu.SemaphoreType.DMA,
            pltpu.SemaphoreType.DMA,
        ],
    )
    def k(table_hbm, gidx_hbm, lrow_hbm, cand_hbm, labc_hbm,
          gidx_v, lrow_v, buf, labbuf, sem_c, sem_l):
        wid = lax.axis_index("s") * nc + lax.axis_index("c")
        pltpu.sync_copy(gidx_hbm.at[wid], gidx_v)
        pltpu.sync_copy(lrow_hbm.at[wid], lrow_v)
        lab_cp = pltpu.async_copy(table_hbm.at[lrow_v], labbuf, sem_l)
        for j in range(K):
            cp = pltpu.async_copy(table_hbm.at[gidx_v.at[j]], buf, sem_c)
            cp.wait()
            pltpu.sync_copy(
                buf, cand_hbm.at[pl.ds(wid * (K * rpt) + j * rpt, rpt)])
        lab_cp.wait()
        pltpu.sync_copy(labbuf, labc_hbm.at[pl.ds(wid * rpt, rpt)])

    return k(table, gidx3, lrow2)


def _loss(cand4, labc, llane2):
    def body(c_ref, lc_ref, ll_ref, o_ref):
        x = c_ref[...]
        colio = lax.broadcasted_iota(jnp.int32, (_BM_LOSS, K * C), 1)
        vals = []
        for kk in range(K):
            mx = jnp.max(x, axis=1, keepdims=True)
            vals.append(mx)
            if kk < K - 1:
                idx = jnp.min(jnp.where(x == mx, colio, K * C), axis=1,
                              keepdims=True)
                x = jnp.where(colio == idx, NEG, x)
        m1 = vals[0]
        s = jnp.exp(vals[0] - m1)
        for v in vals[1:]:
            s = s + jnp.exp(v - m1)
        lse = m1 + jnp.log(s)
        lanio = lax.broadcasted_iota(jnp.int32, (_BM_LOSS, C), 1)
        lv = jnp.sum(jnp.where(lanio == ll_ref[...], lc_ref[...], 0.0),
                     axis=1, keepdims=True)
        part = jnp.sum(lse - lv) * (1.0 / B)

        @pl.when(pl.program_id(0) == 0)
        def _init():
            o_ref[0, 0] = 0.0

        o_ref[0, 0] += part

    return pl.pallas_call(
        body,
        grid=(B // _BM_LOSS,),
        in_specs=[pl.BlockSpec((_BM_LOSS, K * C), lambda i: (i, 0)),
                  pl.BlockSpec((_BM_LOSS, C), lambda i: (i, 0)),
                  pl.BlockSpec((_BM_LOSS, 1), lambda i: (i, 0))],
        out_specs=pl.BlockSpec((1, 1), lambda i: (0, 0),
                               memory_space=pltpu.SMEM),
        out_shape=jax.ShapeDtypeStruct((1, 1), jnp.float32),
    )(cand4, labc, llane2)


def kernel(preds, labels):
    preds = preds.reshape(B, V)
    labels = labels.astype(jnp.int32)
    flat = preds.reshape(FLAT, C)

    m = _chunk_max(flat)
    gidx, lrow, llane = _select_chunks(m.reshape(B, NCHUNK),
                                       labels.reshape(B, 1))

    info = plsc.get_sparse_core_info()
    nc, ns = info.num_cores, info.num_subcores
    nw = nc * ns
    rpt = B // nw
    cand, labc = _sc_gather(
        flat,
        gidx.reshape(nw, K, rpt),
        lrow.reshape(nw, rpt),
        nc, ns)

    out = _loss(cand.reshape(B, K * C), labc, llane)
    return out[0, 0]
